# baseline (speedup 1.0000x reference)
import functools

import jax
import jax.numpy as jnp
from jax import lax
from jax.experimental import pallas as pl
from jax.experimental.pallas import tpu as pltpu
from jax.experimental.pallas import tpu_sc as plsc

BATCH = 1024
CTX = 20
EMBED = 128
LANES = 16
NW = 32
ROWS_PER_W = BATCH // NW
IDX_PER_W = ROWS_PER_W * CTX
IDX_CHUNK = 128
N_CHUNKS = IDX_PER_W // IDX_CHUNK

VT = 1024


def _gather_sum_body(idx_hbm, table_hbm, out_hbm, idx_v, rows_v, acc_v, sem):
  wid = lax.axis_index("s") * 2 + lax.axis_index("c")
  pltpu.sync_copy(idx_hbm.at[pl.ds(wid * IDX_PER_W, IDX_PER_W)], idx_v)
  copies = []
  for c in range(N_CHUNKS):
    copies.append(
        pltpu.make_async_copy(
            table_hbm.at[idx_v.at[pl.ds(c * IDX_CHUNK, IDX_CHUNK)]],
            rows_v.at[pl.ds(c * IDX_CHUNK, IDX_CHUNK)],
            sem,
        )
    )
  for cp in copies:
    cp.start()
  for cp in copies:
    cp.wait()

  def row_body(r, _):
    for k in range(EMBED // LANES):
      acc = rows_v[r * CTX, pl.ds(k * LANES, LANES)]
      for j in range(1, CTX):
        acc = acc + rows_v[r * CTX + j, pl.ds(k * LANES, LANES)]
      acc_v[r, pl.ds(k * LANES, LANES)] = acc
    return 0

  lax.fori_loop(0, ROWS_PER_W, row_body, 0)
  pltpu.sync_copy(acc_v, out_hbm.at[pl.ds(wid * ROWS_PER_W, ROWS_PER_W)])


def _embed_sum(context, emb_table):
  idx1d = context.reshape(BATCH * CTX).astype(jnp.int32)
  mesh = plsc.VectorSubcoreMesh(core_axis_name="c", subcore_axis_name="s")
  f = pl.kernel(
      _gather_sum_body,
      out_type=jax.ShapeDtypeStruct((BATCH, EMBED), jnp.float32),
      mesh=mesh,
      scratch_types=[
          pltpu.VMEM((IDX_PER_W,), jnp.int32),
          pltpu.VMEM((IDX_PER_W, EMBED), jnp.float32),
          pltpu.VMEM((ROWS_PER_W, EMBED), jnp.float32),
          pltpu.SemaphoreType.DMA,
      ],
  )
  return f(idx1d, emb_table)


def _matmul_body(x_ref, w_ref, b_ref, o_ref):
  acc = lax.dot_general(
      x_ref[...], w_ref[...],
      dimension_numbers=(((1,), (1,)), ((), ())),
      preferred_element_type=jnp.float32,
  )
  o_ref[...] = acc + b_ref[...]


def _project(x, W, b):
  vocab = W.shape[0]
  nvt = pl.cdiv(vocab, VT)
  b2d = b.reshape(1, vocab)
  return pl.pallas_call(
      _matmul_body,
      grid=(nvt,),
      in_specs=[
          pl.BlockSpec((BATCH, EMBED), lambda i: (0, 0)),
          pl.BlockSpec((VT, EMBED), lambda i: (i, 0)),
          pl.BlockSpec((1, VT), lambda i: (0, i)),
      ],
      out_specs=pl.BlockSpec((BATCH, VT), lambda i: (0, i)),
      out_shape=jax.ShapeDtypeStruct((BATCH, vocab), jnp.float32),
  )(x, W, b2d)


@jax.jit
def kernel(context, emb_table, W, b):
  x = _embed_sum(context, emb_table)
  return _project(x, W, b)
